# baseline (speedup 1.0000x reference)
import functools

import jax
import jax.numpy as jnp
from jax import lax
from jax.experimental import pallas as pl
from jax.experimental.pallas import tpu as pltpu
from jax.experimental.pallas import tpu_sc as plsc

_TPB = 96
_NSUB = 16


@functools.lru_cache(maxsize=None)
def _make_segsum(n, nch, nb, ncall=None, chunk0=0):
    if ncall is None:
        ncall = nch
    r_slab = n + 112
    rpt = n // _NSUB
    n_it = ncall // 2
    mesh = plsc.VectorSubcoreMesh(core_axis_name="c", subcore_axis_name="s")

    nb2 = nb // 2
    assert nb2 * 2 == nb

    def body(h3, zeros, srcp, dstp, out, src_v, gidx_v, rows_a, rows_b,
             slab, sema, semb, semc, semd):
        cid = lax.axis_index("c")
        sid = lax.axis_index("s")
        pltpu.sync_copy(srcp.at[sid], src_v)
        pltpu.sync_copy(dstp.at[sid], gidx_v)

        def mk0(j, carry):
            for k in range(_TPB // 16):
                sl = pl.ds(k * 16, 16)
                gidx_v[j, sl] = gidx_v[j, sl] * nch + (chunk0 + cid * n_it)
            return carry

        lax.fori_loop(0, nb, mk0, 0)
        for it in range(n_it):
            if it > 0:
                def mk1(j, carry):
                    for k in range(_TPB // 16):
                        sl = pl.ds(k * 16, 16)
                        gidx_v[j, sl] = gidx_v[j, sl] + 1
                    return carry

                lax.fori_loop(0, nb, mk1, 0)
            chunk = cid * n_it + it
            pltpu.sync_copy(zeros, slab.at[pl.ds(sid * rpt, rpt)])
            plsc.subcore_barrier()

            pltpu.async_copy(h3.at[gidx_v.at[0]], rows_a, sema)

            def batch2(jj, carry):
                j0 = jj * 2
                j1 = j0 + 1
                pltpu.make_async_copy(h3.at[gidx_v.at[j0]], rows_a,
                                      sema).wait()
                pltpu.async_copy(h3.at[gidx_v.at[j1]], rows_b, semb)
                pltpu.sync_copy(rows_a, slab.at[src_v.at[j0]], add=True)
                pltpu.make_async_copy(h3.at[gidx_v.at[j1]], rows_b,
                                      semb).wait()

                @pl.when(jj + 1 < nb2)
                def _():
                    pltpu.async_copy(h3.at[gidx_v.at[j0 + 2]], rows_a, sema)

                pltpu.sync_copy(rows_b, slab.at[src_v.at[j1]], add=True)
                return carry

            lax.fori_loop(0, nb2, batch2, 0)
            plsc.subcore_barrier()
            pltpu.sync_copy(slab.at[pl.ds(sid * rpt, rpt)],
                            out.at[chunk, pl.ds(sid * rpt, rpt)])

    return pl.kernel(
        body,
        out_type=jax.ShapeDtypeStruct((ncall, n, 128), jnp.float32),
        mesh=mesh,
        compiler_params=pltpu.CompilerParams(use_tc_tiling_on_sc=False),
        scratch_types=[
            pltpu.VMEM((nb, _TPB), jnp.int32),
            pltpu.VMEM((nb, _TPB), jnp.int32),
            pltpu.VMEM((_TPB, 128), jnp.float32),
            pltpu.VMEM((_TPB, 128), jnp.float32),
            pltpu.VMEM_SHARED((r_slab, 128), jnp.float32),
            pltpu.SemaphoreType.DMA,
            pltpu.SemaphoreType.DMA,
            pltpu.SemaphoreType.DMA,
            pltpu.SemaphoreType.DMA,
        ],
    )


def _dot(a, b):
    return lax.dot_general(a, b, (((1,), (0,)), ((), ())),
                           preferred_element_type=jnp.float32)


@functools.lru_cache(maxsize=None)
def _make_mm_stats(nch, n, hid, rb):
    def kern(p_ref, h_ref, w_ref, b_ref, m_ref, st_ref):
        i = pl.program_id(0)
        acc = jnp.zeros((rb, hid), jnp.float32)
        for c in range(nch):
            acc = acc + _dot(p_ref[c] + h_ref[:, c * 128:(c + 1) * 128],
                             w_ref[c])
        m = acc + b_ref[...]
        m_ref[...] = m

        @pl.when(i == 0)
        def _():
            st_ref[...] = jnp.zeros_like(st_ref)

        st_ref[...] += jnp.concatenate(
            [jnp.sum(m, 0, keepdims=True), jnp.sum(m * m, 0, keepdims=True)], 0)

    return pl.pallas_call(
        kern,
        grid=(n // rb,),
        in_specs=[
            pl.BlockSpec((nch, rb, 128), lambda i: (0, i, 0)),
            pl.BlockSpec((rb, nch * 128), lambda i: (i, 0)),
            pl.BlockSpec((nch, 128, hid), lambda i: (0, 0, 0)),
            pl.BlockSpec((1, hid), lambda i: (0, 0)),
        ],
        out_specs=[
            pl.BlockSpec((rb, hid), lambda i: (i, 0)),
            pl.BlockSpec((2, hid), lambda i: (0, 0)),
        ],
        out_shape=[
            jax.ShapeDtypeStruct((n, hid), jnp.float32),
            jax.ShapeDtypeStruct((2, hid), jnp.float32),
        ],
    )


@functools.lru_cache(maxsize=None)
def _make_mm_half1(n, hid, rb):
    def kern(p_ref, h_ref, w_ref, macc_ref):
        acc = _dot(p_ref[0] + h_ref[:, 0:128], w_ref[0])
        macc_ref[...] = acc + _dot(p_ref[1] + h_ref[:, 128:256], w_ref[1])

    return pl.pallas_call(
        kern,
        grid=(n // rb,),
        in_specs=[
            pl.BlockSpec((2, rb, 128), lambda i: (0, i, 0)),
            pl.BlockSpec((rb, 256), lambda i: (i, 0)),
            pl.BlockSpec((2, 128, hid), lambda i: (0, 0, 0)),
        ],
        out_specs=pl.BlockSpec((rb, hid), lambda i: (i, 0)),
        out_shape=jax.ShapeDtypeStruct((n, hid), jnp.float32),
    )


@functools.lru_cache(maxsize=None)
def _make_mm_half2(n, hid, rb):
    def kern(p_ref, h_ref, macc_ref, w_ref, b_ref, m_ref, st_ref):
        i = pl.program_id(0)
        acc = macc_ref[...] + _dot(p_ref[0] + h_ref[:, 0:128], w_ref[0])
        acc = acc + _dot(p_ref[1] + h_ref[:, 128:256], w_ref[1])
        m = acc + b_ref[...]
        m_ref[...] = m

        @pl.when(i == 0)
        def _():
            st_ref[...] = jnp.zeros_like(st_ref)

        st_ref[...] += jnp.concatenate(
            [jnp.sum(m, 0, keepdims=True), jnp.sum(m * m, 0, keepdims=True)], 0)

    return pl.pallas_call(
        kern,
        grid=(n // rb,),
        in_specs=[
            pl.BlockSpec((2, rb, 128), lambda i: (0, i, 0)),
            pl.BlockSpec((rb, 256), lambda i: (i, 1)),
            pl.BlockSpec((rb, hid), lambda i: (i, 0)),
            pl.BlockSpec((2, 128, hid), lambda i: (0, 0, 0)),
            pl.BlockSpec((1, hid), lambda i: (0, 0)),
        ],
        out_specs=[
            pl.BlockSpec((rb, hid), lambda i: (i, 0)),
            pl.BlockSpec((2, hid), lambda i: (0, 0)),
        ],
        out_shape=[
            jax.ShapeDtypeStruct((n, hid), jnp.float32),
            jax.ShapeDtypeStruct((2, hid), jnp.float32),
        ],
    )


@functools.lru_cache(maxsize=None)
def _make_bn_mm(n, hid, rb):
    def kern(m_ref, st_ref, g_ref, bb_ref, w_ref, b2_ref, y_ref, yst_ref):
        i = pl.program_id(0)
        mean = st_ref[0:1] * (1.0 / n)
        var = st_ref[1:2] * (1.0 / n) - mean * mean
        scale = g_ref[...] * lax.rsqrt(var + 1e-5)
        t = jnp.maximum((m_ref[...] - mean) * scale + bb_ref[...], 0.0)
        y = _dot(t, w_ref[...]) + b2_ref[...]
        y_ref[...] = y

        @pl.when(i == 0)
        def _():
            yst_ref[...] = jnp.zeros_like(yst_ref)

        yst_ref[...] += jnp.concatenate(
            [jnp.sum(y, 0, keepdims=True), jnp.sum(y * y, 0, keepdims=True)], 0)

    return pl.pallas_call(
        kern,
        grid=(n // rb,),
        in_specs=[
            pl.BlockSpec((rb, hid), lambda i: (i, 0)),
            pl.BlockSpec((2, hid), lambda i: (0, 0)),
            pl.BlockSpec((1, hid), lambda i: (0, 0)),
            pl.BlockSpec((1, hid), lambda i: (0, 0)),
            pl.BlockSpec((hid, hid), lambda i: (0, 0)),
            pl.BlockSpec((1, hid), lambda i: (0, 0)),
        ],
        out_specs=[
            pl.BlockSpec((rb, hid), lambda i: (i, 0)),
            pl.BlockSpec((2, hid), lambda i: (0, 0)),
        ],
        out_shape=[
            jax.ShapeDtypeStruct((n, hid), jnp.float32),
            jax.ShapeDtypeStruct((2, hid), jnp.float32),
        ],
    )


@functools.lru_cache(maxsize=None)
def _make_bn_relu(n, hid, rb):
    def kern(y_ref, st_ref, g_ref, bb_ref, h_ref, hs_ref):
        i = pl.program_id(0)
        mean = st_ref[0:1] * (1.0 / n)
        var = st_ref[1:2] * (1.0 / n) - mean * mean
        scale = g_ref[...] * lax.rsqrt(var + 1e-5)
        h = jnp.maximum((y_ref[...] - mean) * scale + bb_ref[...], 0.0)
        h_ref[...] = h

        @pl.when(i == 0)
        def _():
            hs_ref[...] = jnp.zeros_like(hs_ref)

        hs_ref[...] += jnp.sum(h, 0, keepdims=True)

    return pl.pallas_call(
        kern,
        grid=(n // rb,),
        in_specs=[
            pl.BlockSpec((rb, hid), lambda i: (i, 0)),
            pl.BlockSpec((2, hid), lambda i: (0, 0)),
            pl.BlockSpec((1, hid), lambda i: (0, 0)),
            pl.BlockSpec((1, hid), lambda i: (0, 0)),
        ],
        out_specs=[
            pl.BlockSpec((rb, hid), lambda i: (i, 0)),
            pl.BlockSpec((1, hid), lambda i: (0, 0)),
        ],
        out_shape=[
            jax.ShapeDtypeStruct((n, hid), jnp.float32),
            jax.ShapeDtypeStruct((1, hid), jnp.float32),
        ],
    )


@functools.lru_cache(maxsize=None)
def _make_colsum(n, d, rb):
    def kern(x_ref, s_ref):
        i = pl.program_id(0)

        @pl.when(i == 0)
        def _():
            s_ref[...] = jnp.zeros_like(s_ref)

        s_ref[...] += jnp.sum(x_ref[...], 0, keepdims=True)

    return pl.pallas_call(
        kern,
        grid=(n // rb,),
        in_specs=[pl.BlockSpec((rb, d), lambda i: (i, 0))],
        out_specs=pl.BlockSpec((1, d), lambda i: (0, 0)),
        out_shape=jax.ShapeDtypeStruct((1, d), jnp.float32),
    )


@functools.lru_cache(maxsize=None)
def _make_final(din, hid, odim, nl):
    def kern(xs_ref, p0_ref, hs_ref, pw_ref, pb_ref, o_ref):
        acc = _dot(xs_ref[...], p0_ref[...])
        for l in range(nl):
            acc = acc + _dot(hs_ref[l:l + 1], pw_ref[l])
        o_ref[...] = acc + jnp.sum(pb_ref[...], 0, keepdims=True)

    return pl.pallas_call(
        kern,
        out_shape=jax.ShapeDtypeStruct((1, odim), jnp.float32),
    )


def kernel(x, edge_index, batch, params):
    n, din = x.shape
    hid = params['convs'][0]['W1'].shape[1]
    odim = params['preds'][0]['W'].shape[1]
    e = edge_index.shape[1]
    nl = len(params['convs'])
    rb = 1000

    nb = -(-e // (_NSUB * _TPB))
    nb += nb & 1
    pad = _NSUB * _TPB * nb - e
    ar = jnp.arange(pad, dtype=jnp.int32)
    src = jnp.concatenate([edge_index[0], n + ar % 96])
    dst = jnp.concatenate([edge_index[1], (ar * 37) % n])
    srcp = src.reshape(_NSUB, nb, _TPB)
    dstp = dst.reshape(_NSUB, nb, _TPB)
    zeros = jnp.zeros((n // _NSUB, 128), jnp.float32)

    hs_list = []
    h = x
    for l in range(nl):
        c = params['convs'][l]
        nch = h.shape[1] // 128
        h3 = h.reshape(n * nch, 128)
        w1r = c['W1'].reshape(nch, 128, hid)
        b1 = c['b1'].reshape(1, hid)
        if nch == 2:
            pooled = _make_segsum(n, nch, nb)(h3, zeros, srcp, dstp)
            m, mst = _make_mm_stats(nch, n, hid, rb)(pooled, h, w1r, b1)
        else:
            p01 = _make_segsum(n, nch, nb, 2, 0)(h3, zeros, srcp, dstp)
            p23 = _make_segsum(n, nch, nb, 2, 2)(h3, zeros, srcp, dstp)
            macc = _make_mm_half1(n, hid, rb)(p01, h, w1r[0:2])
            m, mst = _make_mm_half2(n, hid, rb)(p23, h, macc, w1r[2:4], b1)
        y, yst = _make_bn_mm(n, hid, rb)(
            m, mst, c['bn1_g'].reshape(1, hid), c['bn1_b'].reshape(1, hid),
            c['W2'], c['b2'].reshape(1, hid))
        h, hsum = _make_bn_relu(n, hid, rb)(
            y, yst, c['bn_g'].reshape(1, hid), c['bn_b'].reshape(1, hid))
        hs_list.append(hsum)

    xsum = _make_colsum(n, din, rb)(x)
    hs = jnp.concatenate(hs_list, 0)
    pw = jnp.stack([params['preds'][l + 1]['W'] for l in range(nl)])
    pb = jnp.stack([params['preds'][l]['b'].reshape(1, odim)
                    for l in range(nl + 1)]).reshape(nl + 1, odim)
    return _make_final(din, hid, odim, nl)(
        xsum, params['preds'][0]['W'], hs, pw, pb)


# speedup vs baseline: 1.0343x; 1.0343x over previous
import functools

import jax
import jax.numpy as jnp
from jax import lax
from jax.experimental import pallas as pl
from jax.experimental.pallas import tpu as pltpu
from jax.experimental.pallas import tpu_sc as plsc

_TPB = 96
_NSUB = 16


@functools.lru_cache(maxsize=None)
def _make_segsum(n, nch, nb, ncall=None, chunk0=0):
    if ncall is None:
        ncall = nch
    r_slab = n + 112
    rpt = n // _NSUB
    n_it = ncall // 2
    mesh = plsc.VectorSubcoreMesh(core_axis_name="c", subcore_axis_name="s")

    nb2 = nb // 2
    assert nb2 * 2 == nb

    def body(h3, zeros, srcp, dstp, out, src_v, gidx_v, rows_a, rows_b,
             slab, sema, semb, semc, semd):
        cid = lax.axis_index("c")
        sid = lax.axis_index("s")
        pltpu.sync_copy(srcp.at[sid], src_v)
        pltpu.sync_copy(dstp.at[sid], gidx_v)

        def mk0(j, carry):
            for k in range(_TPB // 16):
                sl = pl.ds(k * 16, 16)
                gidx_v[j, sl] = gidx_v[j, sl] * nch + (chunk0 + cid * n_it)
            return carry

        lax.fori_loop(0, nb, mk0, 0)
        for it in range(n_it):
            if it > 0:
                def mk1(j, carry):
                    for k in range(_TPB // 16):
                        sl = pl.ds(k * 16, 16)
                        gidx_v[j, sl] = gidx_v[j, sl] + 1
                    return carry

                lax.fori_loop(0, nb, mk1, 0)
            chunk = cid * n_it + it
            pltpu.sync_copy(zeros, slab.at[pl.ds(sid * rpt, rpt)])
            plsc.subcore_barrier()

            pltpu.async_copy(h3.at[gidx_v.at[0]], rows_a, sema)

            def batch2(jj, carry):
                j0 = jj * 2
                j1 = j0 + 1
                pltpu.make_async_copy(h3.at[gidx_v.at[j0]], rows_a,
                                      sema).wait()
                pltpu.async_copy(h3.at[gidx_v.at[j1]], rows_b, semb)
                pltpu.sync_copy(rows_a, slab.at[src_v.at[j0]], add=True)
                pltpu.make_async_copy(h3.at[gidx_v.at[j1]], rows_b,
                                      semb).wait()

                @pl.when(jj + 1 < nb2)
                def _():
                    pltpu.async_copy(h3.at[gidx_v.at[j0 + 2]], rows_a, sema)

                pltpu.sync_copy(rows_b, slab.at[src_v.at[j1]], add=True)
                return carry

            lax.fori_loop(0, nb2, batch2, 0)
            plsc.subcore_barrier()
            pltpu.sync_copy(slab.at[pl.ds(sid * rpt, rpt)],
                            out.at[chunk, pl.ds(sid * rpt, rpt)])

    return pl.kernel(
        body,
        out_type=jax.ShapeDtypeStruct((ncall, n, 128), jnp.float32),
        mesh=mesh,
        compiler_params=pltpu.CompilerParams(use_tc_tiling_on_sc=False),
        scratch_types=[
            pltpu.VMEM((nb, _TPB), jnp.int32),
            pltpu.VMEM((nb, _TPB), jnp.int32),
            pltpu.VMEM((_TPB, 128), jnp.float32),
            pltpu.VMEM((_TPB, 128), jnp.float32),
            pltpu.VMEM_SHARED((r_slab, 128), jnp.float32),
            pltpu.SemaphoreType.DMA,
            pltpu.SemaphoreType.DMA,
            pltpu.SemaphoreType.DMA,
            pltpu.SemaphoreType.DMA,
        ],
    )


def _dot(a, b):
    return lax.dot_general(a, b, (((1,), (0,)), ((), ())),
                           preferred_element_type=jnp.float32)


@functools.lru_cache(maxsize=None)
def _make_mm_stats(nch, n, hid, rb):
    def kern(p_ref, h_ref, w_ref, b_ref, m_ref, st_ref):
        i = pl.program_id(0)
        acc = jnp.zeros((rb, hid), jnp.float32)
        for c in range(nch):
            acc = acc + _dot(p_ref[c] + h_ref[:, c * 128:(c + 1) * 128],
                             w_ref[c])
        m = acc + b_ref[...]
        m_ref[...] = m

        @pl.when(i == 0)
        def _():
            st_ref[...] = jnp.zeros_like(st_ref)

        st_ref[...] += jnp.concatenate(
            [jnp.sum(m, 0, keepdims=True), jnp.sum(m * m, 0, keepdims=True)], 0)

    return pl.pallas_call(
        kern,
        grid=(n // rb,),
        in_specs=[
            pl.BlockSpec((nch, rb, 128), lambda i: (0, i, 0)),
            pl.BlockSpec((rb, nch * 128), lambda i: (i, 0)),
            pl.BlockSpec((nch, 128, hid), lambda i: (0, 0, 0)),
            pl.BlockSpec((1, hid), lambda i: (0, 0)),
        ],
        out_specs=[
            pl.BlockSpec((rb, hid), lambda i: (i, 0)),
            pl.BlockSpec((2, hid), lambda i: (0, 0)),
        ],
        out_shape=[
            jax.ShapeDtypeStruct((n, hid), jnp.float32),
            jax.ShapeDtypeStruct((2, hid), jnp.float32),
        ],
    )


@functools.lru_cache(maxsize=None)
def _make_bn_mm(n, hid, rb):
    def kern(m_ref, st_ref, g_ref, bb_ref, w_ref, b2_ref, y_ref, yst_ref):
        i = pl.program_id(0)
        mean = st_ref[0:1] * (1.0 / n)
        var = st_ref[1:2] * (1.0 / n) - mean * mean
        scale = g_ref[...] * lax.rsqrt(var + 1e-5)
        t = jnp.maximum((m_ref[...] - mean) * scale + bb_ref[...], 0.0)
        y = _dot(t, w_ref[...]) + b2_ref[...]
        y_ref[...] = y

        @pl.when(i == 0)
        def _():
            yst_ref[...] = jnp.zeros_like(yst_ref)

        yst_ref[...] += jnp.concatenate(
            [jnp.sum(y, 0, keepdims=True), jnp.sum(y * y, 0, keepdims=True)], 0)

    return pl.pallas_call(
        kern,
        grid=(n // rb,),
        in_specs=[
            pl.BlockSpec((rb, hid), lambda i: (i, 0)),
            pl.BlockSpec((2, hid), lambda i: (0, 0)),
            pl.BlockSpec((1, hid), lambda i: (0, 0)),
            pl.BlockSpec((1, hid), lambda i: (0, 0)),
            pl.BlockSpec((hid, hid), lambda i: (0, 0)),
            pl.BlockSpec((1, hid), lambda i: (0, 0)),
        ],
        out_specs=[
            pl.BlockSpec((rb, hid), lambda i: (i, 0)),
            pl.BlockSpec((2, hid), lambda i: (0, 0)),
        ],
        out_shape=[
            jax.ShapeDtypeStruct((n, hid), jnp.float32),
            jax.ShapeDtypeStruct((2, hid), jnp.float32),
        ],
    )


@functools.lru_cache(maxsize=None)
def _make_bn_relu(n, hid, rb):
    def kern(y_ref, st_ref, g_ref, bb_ref, h_ref, hs_ref):
        i = pl.program_id(0)
        mean = st_ref[0:1] * (1.0 / n)
        var = st_ref[1:2] * (1.0 / n) - mean * mean
        scale = g_ref[...] * lax.rsqrt(var + 1e-5)
        h = jnp.maximum((y_ref[...] - mean) * scale + bb_ref[...], 0.0)
        h_ref[...] = h

        @pl.when(i == 0)
        def _():
            hs_ref[...] = jnp.zeros_like(hs_ref)

        hs_ref[...] += jnp.sum(h, 0, keepdims=True)

    return pl.pallas_call(
        kern,
        grid=(n // rb,),
        in_specs=[
            pl.BlockSpec((rb, hid), lambda i: (i, 0)),
            pl.BlockSpec((2, hid), lambda i: (0, 0)),
            pl.BlockSpec((1, hid), lambda i: (0, 0)),
            pl.BlockSpec((1, hid), lambda i: (0, 0)),
        ],
        out_specs=[
            pl.BlockSpec((rb, hid), lambda i: (i, 0)),
            pl.BlockSpec((1, hid), lambda i: (0, 0)),
        ],
        out_shape=[
            jax.ShapeDtypeStruct((n, hid), jnp.float32),
            jax.ShapeDtypeStruct((1, hid), jnp.float32),
        ],
    )


@functools.lru_cache(maxsize=None)
def _make_bn_sum(n, hid, rb):
    def kern(y_ref, st_ref, g_ref, bb_ref, hs_ref):
        i = pl.program_id(0)
        mean = st_ref[0:1] * (1.0 / n)
        var = st_ref[1:2] * (1.0 / n) - mean * mean
        scale = g_ref[...] * lax.rsqrt(var + 1e-5)
        h = jnp.maximum((y_ref[...] - mean) * scale + bb_ref[...], 0.0)

        @pl.when(i == 0)
        def _():
            hs_ref[...] = jnp.zeros_like(hs_ref)

        hs_ref[...] += jnp.sum(h, 0, keepdims=True)

    return pl.pallas_call(
        kern,
        grid=(n // rb,),
        in_specs=[
            pl.BlockSpec((rb, hid), lambda i: (i, 0)),
            pl.BlockSpec((2, hid), lambda i: (0, 0)),
            pl.BlockSpec((1, hid), lambda i: (0, 0)),
            pl.BlockSpec((1, hid), lambda i: (0, 0)),
        ],
        out_specs=pl.BlockSpec((1, hid), lambda i: (0, 0)),
        out_shape=jax.ShapeDtypeStruct((1, hid), jnp.float32),
    )


@functools.lru_cache(maxsize=None)
def _make_colsum(n, d, rb):
    def kern(x_ref, s_ref):
        i = pl.program_id(0)

        @pl.when(i == 0)
        def _():
            s_ref[...] = jnp.zeros_like(s_ref)

        s_ref[...] += jnp.sum(x_ref[...], 0, keepdims=True)

    return pl.pallas_call(
        kern,
        grid=(n // rb,),
        in_specs=[pl.BlockSpec((rb, d), lambda i: (i, 0))],
        out_specs=pl.BlockSpec((1, d), lambda i: (0, 0)),
        out_shape=jax.ShapeDtypeStruct((1, d), jnp.float32),
    )


@functools.lru_cache(maxsize=None)
def _make_final(din, hid, odim, nl):
    def kern(xs_ref, p0_ref, hs_ref, pw_ref, pb_ref, o_ref):
        acc = _dot(xs_ref[...], p0_ref[...])
        for l in range(nl):
            acc = acc + _dot(hs_ref[l:l + 1], pw_ref[l])
        o_ref[...] = acc + jnp.sum(pb_ref[...], 0, keepdims=True)

    return pl.pallas_call(
        kern,
        out_shape=jax.ShapeDtypeStruct((1, odim), jnp.float32),
    )


def kernel(x, edge_index, batch, params):
    n, din = x.shape
    hid = params['convs'][0]['W1'].shape[1]
    odim = params['preds'][0]['W'].shape[1]
    e = edge_index.shape[1]
    nl = len(params['convs'])
    rb = 1000

    nb = -(-e // (_NSUB * _TPB))
    nb += nb & 1
    pad = _NSUB * _TPB * nb - e
    ar = jnp.arange(pad, dtype=jnp.int32)
    src = jnp.concatenate([edge_index[0], n + ar % 96])
    dst = jnp.concatenate([edge_index[1], (ar * 37) % n])
    srcp = src.reshape(_NSUB, nb, _TPB)
    dstp = dst.reshape(_NSUB, nb, _TPB)
    zeros = jnp.zeros((n // _NSUB, 128), jnp.float32)

    hs_list = []
    h = x
    for l in range(nl):
        c = params['convs'][l]
        nch = h.shape[1] // 128
        h3 = h.reshape(n * nch, 128)
        w1r = c['W1'].reshape(nch, 128, hid)
        b1 = c['b1'].reshape(1, hid)
        pooled = _make_segsum(n, nch, nb)(h3, zeros, srcp, dstp)
        m, mst = _make_mm_stats(nch, n, hid, rb)(pooled, h, w1r, b1)
        y, yst = _make_bn_mm(n, hid, rb)(
            m, mst, c['bn1_g'].reshape(1, hid), c['bn1_b'].reshape(1, hid),
            c['W2'], c['b2'].reshape(1, hid))
        if l + 1 < nl:
            h, hsum = _make_bn_relu(n, hid, rb)(
                y, yst, c['bn_g'].reshape(1, hid), c['bn_b'].reshape(1, hid))
        else:
            hsum = _make_bn_sum(n, hid, rb)(
                y, yst, c['bn_g'].reshape(1, hid), c['bn_b'].reshape(1, hid))
        hs_list.append(hsum)

    xsum = _make_colsum(n, din, rb)(x)
    hs = jnp.concatenate(hs_list, 0)
    pw = jnp.stack([params['preds'][l + 1]['W'] for l in range(nl)])
    pb = jnp.stack([params['preds'][l]['b'].reshape(1, odim)
                    for l in range(nl + 1)]).reshape(nl + 1, odim)
    return _make_final(din, hid, odim, nl)(
        xsum, params['preds'][0]['W'], hs, pw, pb)


# speedup vs baseline: 1.0466x; 1.0119x over previous
import functools

import jax
import jax.numpy as jnp
from jax import lax
from jax.experimental import pallas as pl
from jax.experimental.pallas import tpu as pltpu
from jax.experimental.pallas import tpu_sc as plsc

_TPB = 96
_NSUB = 16


@functools.lru_cache(maxsize=None)
def _make_segsum(n, nch, nb, ncall=None, chunk0=0):
    if ncall is None:
        ncall = nch
    r_slab = n + 112
    rpt = n // _NSUB
    n_it = ncall // 2
    mesh = plsc.VectorSubcoreMesh(core_axis_name="c", subcore_axis_name="s")

    nb2 = nb // 2
    assert nb2 * 2 == nb

    def body(h3, zeros, srcp, dstp, out, src_v, gidx_v, rows_a, rows_b,
             slab, sema, semb, semc, semd):
        cid = lax.axis_index("c")
        sid = lax.axis_index("s")
        pltpu.sync_copy(srcp.at[sid], src_v)
        pltpu.sync_copy(dstp.at[sid], gidx_v)

        def mk0(j, carry):
            for k in range(_TPB // 16):
                sl = pl.ds(k * 16, 16)
                gidx_v[j, sl] = gidx_v[j, sl] * nch + (chunk0 + cid * n_it)
            return carry

        lax.fori_loop(0, nb, mk0, 0)
        for it in range(n_it):
            if it > 0:
                def mk1(j, carry):
                    for k in range(_TPB // 16):
                        sl = pl.ds(k * 16, 16)
                        gidx_v[j, sl] = gidx_v[j, sl] + 1
                    return carry

                lax.fori_loop(0, nb, mk1, 0)
            chunk = cid * n_it + it
            pltpu.sync_copy(zeros, slab.at[pl.ds(sid * rpt, rpt)])
            plsc.subcore_barrier()

            pltpu.async_copy(h3.at[gidx_v.at[0]], rows_a, sema)

            def batch2(jj, carry):
                j0 = jj * 2
                j1 = j0 + 1
                pltpu.make_async_copy(h3.at[gidx_v.at[j0]], rows_a,
                                      sema).wait()
                pltpu.async_copy(h3.at[gidx_v.at[j1]], rows_b, semb)
                pltpu.sync_copy(rows_a, slab.at[src_v.at[j0]], add=True)
                pltpu.make_async_copy(h3.at[gidx_v.at[j1]], rows_b,
                                      semb).wait()

                @pl.when(jj + 1 < nb2)
                def _():
                    pltpu.async_copy(h3.at[gidx_v.at[j0 + 2]], rows_a, sema)

                pltpu.sync_copy(rows_b, slab.at[src_v.at[j1]], add=True)
                return carry

            lax.fori_loop(0, nb2, batch2, 0)
            plsc.subcore_barrier()
            pltpu.sync_copy(slab.at[pl.ds(sid * rpt, rpt)],
                            out.at[chunk, pl.ds(sid * rpt, rpt)])

    return pl.kernel(
        body,
        out_type=jax.ShapeDtypeStruct((ncall, n, 128), jnp.float32),
        mesh=mesh,
        compiler_params=pltpu.CompilerParams(use_tc_tiling_on_sc=False),
        scratch_types=[
            pltpu.VMEM((nb, _TPB), jnp.int32),
            pltpu.VMEM((nb, _TPB), jnp.int32),
            pltpu.VMEM((_TPB, 128), jnp.float32),
            pltpu.VMEM((_TPB, 128), jnp.float32),
            pltpu.VMEM_SHARED((r_slab, 128), jnp.float32),
            pltpu.SemaphoreType.DMA,
            pltpu.SemaphoreType.DMA,
            pltpu.SemaphoreType.DMA,
            pltpu.SemaphoreType.DMA,
        ],
    )


def _dot(a, b):
    return lax.dot_general(a, b, (((1,), (0,)), ((), ())),
                           preferred_element_type=jnp.float32)


@functools.lru_cache(maxsize=None)
def _make_mm_stats(nch, n, hid, rb):
    def kern(p_ref, h_ref, w_ref, b_ref, m_ref, st_ref):
        i = pl.program_id(0)
        acc = jnp.zeros((rb, hid), jnp.float32)
        for c in range(nch):
            acc = acc + _dot(p_ref[c] + h_ref[:, c * 128:(c + 1) * 128],
                             w_ref[c])
        m = acc + b_ref[...]
        m_ref[...] = m

        @pl.when(i == 0)
        def _():
            st_ref[...] = jnp.zeros_like(st_ref)

        st_ref[...] += jnp.concatenate(
            [jnp.sum(m, 0, keepdims=True), jnp.sum(m * m, 0, keepdims=True)], 0)

    return pl.pallas_call(
        kern,
        grid=(n // rb,),
        in_specs=[
            pl.BlockSpec((nch, rb, 128), lambda i: (0, i, 0)),
            pl.BlockSpec((rb, nch * 128), lambda i: (i, 0)),
            pl.BlockSpec((nch, 128, hid), lambda i: (0, 0, 0)),
            pl.BlockSpec((1, hid), lambda i: (0, 0)),
        ],
        out_specs=[
            pl.BlockSpec((rb, hid), lambda i: (i, 0)),
            pl.BlockSpec((2, hid), lambda i: (0, 0)),
        ],
        out_shape=[
            jax.ShapeDtypeStruct((n, hid), jnp.float32),
            jax.ShapeDtypeStruct((2, hid), jnp.float32),
        ],
    )


@functools.lru_cache(maxsize=None)
def _make_bn_mm(n, hid, rb):
    def kern(m_ref, st_ref, g_ref, bb_ref, w_ref, b2_ref, y_ref, yst_ref):
        i = pl.program_id(0)
        mean = st_ref[0:1] * (1.0 / n)
        var = st_ref[1:2] * (1.0 / n) - mean * mean
        scale = g_ref[...] * lax.rsqrt(var + 1e-5)
        t = jnp.maximum((m_ref[...] - mean) * scale + bb_ref[...], 0.0)
        y = _dot(t, w_ref[...]) + b2_ref[...]
        y_ref[...] = y

        @pl.when(i == 0)
        def _():
            yst_ref[...] = jnp.zeros_like(yst_ref)

        yst_ref[...] += jnp.concatenate(
            [jnp.sum(y, 0, keepdims=True), jnp.sum(y * y, 0, keepdims=True)], 0)

    return pl.pallas_call(
        kern,
        grid=(n // rb,),
        in_specs=[
            pl.BlockSpec((rb, hid), lambda i: (i, 0)),
            pl.BlockSpec((2, hid), lambda i: (0, 0)),
            pl.BlockSpec((1, hid), lambda i: (0, 0)),
            pl.BlockSpec((1, hid), lambda i: (0, 0)),
            pl.BlockSpec((hid, hid), lambda i: (0, 0)),
            pl.BlockSpec((1, hid), lambda i: (0, 0)),
        ],
        out_specs=[
            pl.BlockSpec((rb, hid), lambda i: (i, 0)),
            pl.BlockSpec((2, hid), lambda i: (0, 0)),
        ],
        out_shape=[
            jax.ShapeDtypeStruct((n, hid), jnp.float32),
            jax.ShapeDtypeStruct((2, hid), jnp.float32),
        ],
    )


@functools.lru_cache(maxsize=None)
def _make_bn_relu(n, hid, rb):
    def kern(y_ref, st_ref, g_ref, bb_ref, h_ref, hs_ref):
        i = pl.program_id(0)
        mean = st_ref[0:1] * (1.0 / n)
        var = st_ref[1:2] * (1.0 / n) - mean * mean
        scale = g_ref[...] * lax.rsqrt(var + 1e-5)
        h = jnp.maximum((y_ref[...] - mean) * scale + bb_ref[...], 0.0)
        h_ref[...] = h

        @pl.when(i == 0)
        def _():
            hs_ref[...] = jnp.zeros_like(hs_ref)

        hs_ref[...] += jnp.sum(h, 0, keepdims=True)

    return pl.pallas_call(
        kern,
        grid=(n // rb,),
        in_specs=[
            pl.BlockSpec((rb, hid), lambda i: (i, 0)),
            pl.BlockSpec((2, hid), lambda i: (0, 0)),
            pl.BlockSpec((1, hid), lambda i: (0, 0)),
            pl.BlockSpec((1, hid), lambda i: (0, 0)),
        ],
        out_specs=[
            pl.BlockSpec((rb, hid), lambda i: (i, 0)),
            pl.BlockSpec((1, hid), lambda i: (0, 0)),
        ],
        out_shape=[
            jax.ShapeDtypeStruct((n, hid), jnp.float32),
            jax.ShapeDtypeStruct((1, hid), jnp.float32),
        ],
    )


@functools.lru_cache(maxsize=None)
def _make_bn_sum(n, hid, rb):
    def kern(y_ref, st_ref, g_ref, bb_ref, hs_ref):
        i = pl.program_id(0)
        mean = st_ref[0:1] * (1.0 / n)
        var = st_ref[1:2] * (1.0 / n) - mean * mean
        scale = g_ref[...] * lax.rsqrt(var + 1e-5)
        h = jnp.maximum((y_ref[...] - mean) * scale + bb_ref[...], 0.0)

        @pl.when(i == 0)
        def _():
            hs_ref[...] = jnp.zeros_like(hs_ref)

        hs_ref[...] += jnp.sum(h, 0, keepdims=True)

    return pl.pallas_call(
        kern,
        grid=(n // rb,),
        in_specs=[
            pl.BlockSpec((rb, hid), lambda i: (i, 0)),
            pl.BlockSpec((2, hid), lambda i: (0, 0)),
            pl.BlockSpec((1, hid), lambda i: (0, 0)),
            pl.BlockSpec((1, hid), lambda i: (0, 0)),
        ],
        out_specs=pl.BlockSpec((1, hid), lambda i: (0, 0)),
        out_shape=jax.ShapeDtypeStruct((1, hid), jnp.float32),
    )


@functools.lru_cache(maxsize=None)
def _make_colsum(n, d, rb):
    def kern(x_ref, s_ref):
        i = pl.program_id(0)

        @pl.when(i == 0)
        def _():
            s_ref[...] = jnp.zeros_like(s_ref)

        s_ref[...] += jnp.sum(x_ref[...], 0, keepdims=True)

    return pl.pallas_call(
        kern,
        grid=(n // rb,),
        in_specs=[pl.BlockSpec((rb, d), lambda i: (i, 0))],
        out_specs=pl.BlockSpec((1, d), lambda i: (0, 0)),
        out_shape=jax.ShapeDtypeStruct((1, d), jnp.float32),
    )


@functools.lru_cache(maxsize=None)
def _make_final(din, hid, odim, nl):
    def kern(xs_ref, p0_ref, hs_ref, pw_ref, pb_ref, o_ref):
        acc = _dot(xs_ref[...], p0_ref[...])
        for l in range(nl):
            acc = acc + _dot(hs_ref[l:l + 1], pw_ref[l])
        o_ref[...] = acc + jnp.sum(pb_ref[...], 0, keepdims=True)

    return pl.pallas_call(
        kern,
        out_shape=jax.ShapeDtypeStruct((1, odim), jnp.float32),
    )


def kernel(x, edge_index, batch, params):
    n, din = x.shape
    hid = params['convs'][0]['W1'].shape[1]
    odim = params['preds'][0]['W'].shape[1]
    e = edge_index.shape[1]
    nl = len(params['convs'])
    rb = 2000

    nb = -(-e // (_NSUB * _TPB))
    nb += nb & 1
    pad = _NSUB * _TPB * nb - e
    ar = jnp.arange(pad, dtype=jnp.int32)
    src = jnp.concatenate([edge_index[0], n + ar % 96])
    dst = jnp.concatenate([edge_index[1], (ar * 37) % n])
    srcp = src.reshape(_NSUB, nb, _TPB)
    dstp = dst.reshape(_NSUB, nb, _TPB)
    zeros = jnp.zeros((n // _NSUB, 128), jnp.float32)

    hs_list = []
    h = x
    for l in range(nl):
        c = params['convs'][l]
        nch = h.shape[1] // 128
        h3 = h.reshape(n * nch, 128)
        w1r = c['W1'].reshape(nch, 128, hid)
        b1 = c['b1'].reshape(1, hid)
        pooled = _make_segsum(n, nch, nb)(h3, zeros, srcp, dstp)
        m, mst = _make_mm_stats(nch, n, hid, rb)(pooled, h, w1r, b1)
        y, yst = _make_bn_mm(n, hid, rb)(
            m, mst, c['bn1_g'].reshape(1, hid), c['bn1_b'].reshape(1, hid),
            c['W2'], c['b2'].reshape(1, hid))
        if l + 1 < nl:
            h, hsum = _make_bn_relu(n, hid, rb)(
                y, yst, c['bn_g'].reshape(1, hid), c['bn_b'].reshape(1, hid))
        else:
            hsum = _make_bn_sum(n, hid, rb)(
                y, yst, c['bn_g'].reshape(1, hid), c['bn_b'].reshape(1, hid))
        hs_list.append(hsum)

    xsum = _make_colsum(n, din, rb)(x)
    hs = jnp.concatenate(hs_list, 0)
    pw = jnp.stack([params['preds'][l + 1]['W'] for l in range(nl)])
    pb = jnp.stack([params['preds'][l]['b'].reshape(1, odim)
                    for l in range(nl + 1)]).reshape(nl + 1, odim)
    return _make_final(din, hid, odim, nl)(
        xsum, params['preds'][0]['W'], hs, pw, pb)


# speedup vs baseline: 1.0475x; 1.0008x over previous
import functools

import jax
import jax.numpy as jnp
from jax import lax
from jax.experimental import pallas as pl
from jax.experimental.pallas import tpu as pltpu
from jax.experimental.pallas import tpu_sc as plsc

_TPB = 96
_NSUB = 16


@functools.lru_cache(maxsize=None)
def _make_segsum(n, nch, nb, ncall=None, chunk0=0):
    if ncall is None:
        ncall = nch
    r_slab = n + 112
    rpt = n // _NSUB
    n_it = ncall // 2
    mesh = plsc.VectorSubcoreMesh(core_axis_name="c", subcore_axis_name="s")

    nb2 = nb // 2
    assert nb2 * 2 == nb

    def body(h3, zeros, srcp, dstp, out, src_v, gidx_v, rows_a, rows_b,
             slab, sema, semb):
        cid = lax.axis_index("c")
        sid = lax.axis_index("s")
        pltpu.sync_copy(srcp.at[sid], src_v)
        pltpu.sync_copy(dstp.at[sid], gidx_v)

        def mk0(j, carry):
            for k in range(_TPB // 16):
                sl = pl.ds(k * 16, 16)
                gidx_v[j, sl] = gidx_v[j, sl] * nch + (chunk0 + cid * n_it)
            return carry

        lax.fori_loop(0, nb, mk0, 0)
        for it in range(n_it):
            if it > 0:
                def mk1(j, carry):
                    for k in range(_TPB // 16):
                        sl = pl.ds(k * 16, 16)
                        gidx_v[j, sl] = gidx_v[j, sl] + 1
                    return carry

                lax.fori_loop(0, nb, mk1, 0)
            chunk = cid * n_it + it
            pltpu.sync_copy(zeros, slab.at[pl.ds(sid * rpt, rpt)])
            plsc.subcore_barrier()

            pltpu.async_copy(h3.at[gidx_v.at[0]], rows_a, sema)

            def batch2(jj, carry):
                j0 = jj * 2
                j1 = j0 + 1
                pltpu.make_async_copy(h3.at[gidx_v.at[j0]], rows_a,
                                      sema).wait()
                pltpu.async_copy(h3.at[gidx_v.at[j1]], rows_b, semb)
                pltpu.sync_copy(rows_a, slab.at[src_v.at[j0]], add=True)
                pltpu.make_async_copy(h3.at[gidx_v.at[j1]], rows_b,
                                      semb).wait()

                @pl.when(jj + 1 < nb2)
                def _():
                    pltpu.async_copy(h3.at[gidx_v.at[j0 + 2]], rows_a, sema)

                pltpu.sync_copy(rows_b, slab.at[src_v.at[j1]], add=True)
                return carry

            lax.fori_loop(0, nb2, batch2, 0)
            plsc.subcore_barrier()
            pltpu.sync_copy(slab.at[pl.ds(sid * rpt, rpt)],
                            out.at[chunk, pl.ds(sid * rpt, rpt)])

    return pl.kernel(
        body,
        out_type=jax.ShapeDtypeStruct((ncall, n, 128), jnp.float32),
        mesh=mesh,
        compiler_params=pltpu.CompilerParams(use_tc_tiling_on_sc=False),
        scratch_types=[
            pltpu.VMEM((nb, _TPB), jnp.int32),
            pltpu.VMEM((nb, _TPB), jnp.int32),
            pltpu.VMEM((_TPB, 128), jnp.float32),
            pltpu.VMEM((_TPB, 128), jnp.float32),
            pltpu.VMEM_SHARED((r_slab, 128), jnp.float32),
            pltpu.SemaphoreType.DMA,
            pltpu.SemaphoreType.DMA,
        ],
    )


def _dot(a, b):
    return lax.dot_general(a, b, (((1,), (0,)), ((), ())),
                           preferred_element_type=jnp.float32)


@functools.lru_cache(maxsize=None)
def _make_mm_stats(nch, n, hid, rb):
    def kern(p_ref, h_ref, w_ref, b_ref, m_ref, st_ref):
        i = pl.program_id(0)
        acc = jnp.zeros((rb, hid), jnp.float32)
        for c in range(nch):
            acc = acc + _dot(p_ref[c] + h_ref[:, c * 128:(c + 1) * 128],
                             w_ref[c])
        m = acc + b_ref[...]
        m_ref[...] = m

        @pl.when(i == 0)
        def _():
            st_ref[...] = jnp.zeros_like(st_ref)

        st_ref[...] += jnp.concatenate(
            [jnp.sum(m, 0, keepdims=True), jnp.sum(m * m, 0, keepdims=True)], 0)

    return pl.pallas_call(
        kern,
        grid=(n // rb,),
        in_specs=[
            pl.BlockSpec((nch, rb, 128), lambda i: (0, i, 0)),
            pl.BlockSpec((rb, nch * 128), lambda i: (i, 0)),
            pl.BlockSpec((nch, 128, hid), lambda i: (0, 0, 0)),
            pl.BlockSpec((1, hid), lambda i: (0, 0)),
        ],
        out_specs=[
            pl.BlockSpec((rb, hid), lambda i: (i, 0)),
            pl.BlockSpec((2, hid), lambda i: (0, 0)),
        ],
        out_shape=[
            jax.ShapeDtypeStruct((n, hid), jnp.float32),
            jax.ShapeDtypeStruct((2, hid), jnp.float32),
        ],
    )


@functools.lru_cache(maxsize=None)
def _make_bn_mm(n, hid, rb):
    def kern(m_ref, st_ref, g_ref, bb_ref, w_ref, b2_ref, y_ref, yst_ref):
        i = pl.program_id(0)
        mean = st_ref[0:1] * (1.0 / n)
        var = st_ref[1:2] * (1.0 / n) - mean * mean
        scale = g_ref[...] * lax.rsqrt(var + 1e-5)
        t = jnp.maximum((m_ref[...] - mean) * scale + bb_ref[...], 0.0)
        y = _dot(t, w_ref[...]) + b2_ref[...]
        y_ref[...] = y

        @pl.when(i == 0)
        def _():
            yst_ref[...] = jnp.zeros_like(yst_ref)

        yst_ref[...] += jnp.concatenate(
            [jnp.sum(y, 0, keepdims=True), jnp.sum(y * y, 0, keepdims=True)], 0)

    return pl.pallas_call(
        kern,
        grid=(n // rb,),
        in_specs=[
            pl.BlockSpec((rb, hid), lambda i: (i, 0)),
            pl.BlockSpec((2, hid), lambda i: (0, 0)),
            pl.BlockSpec((1, hid), lambda i: (0, 0)),
            pl.BlockSpec((1, hid), lambda i: (0, 0)),
            pl.BlockSpec((hid, hid), lambda i: (0, 0)),
            pl.BlockSpec((1, hid), lambda i: (0, 0)),
        ],
        out_specs=[
            pl.BlockSpec((rb, hid), lambda i: (i, 0)),
            pl.BlockSpec((2, hid), lambda i: (0, 0)),
        ],
        out_shape=[
            jax.ShapeDtypeStruct((n, hid), jnp.float32),
            jax.ShapeDtypeStruct((2, hid), jnp.float32),
        ],
    )


@functools.lru_cache(maxsize=None)
def _make_bn_relu(n, hid, rb):
    def kern(y_ref, st_ref, g_ref, bb_ref, h_ref, hs_ref):
        i = pl.program_id(0)
        mean = st_ref[0:1] * (1.0 / n)
        var = st_ref[1:2] * (1.0 / n) - mean * mean
        scale = g_ref[...] * lax.rsqrt(var + 1e-5)
        h = jnp.maximum((y_ref[...] - mean) * scale + bb_ref[...], 0.0)
        h_ref[...] = h

        @pl.when(i == 0)
        def _():
            hs_ref[...] = jnp.zeros_like(hs_ref)

        hs_ref[...] += jnp.sum(h, 0, keepdims=True)

    return pl.pallas_call(
        kern,
        grid=(n // rb,),
        in_specs=[
            pl.BlockSpec((rb, hid), lambda i: (i, 0)),
            pl.BlockSpec((2, hid), lambda i: (0, 0)),
            pl.BlockSpec((1, hid), lambda i: (0, 0)),
            pl.BlockSpec((1, hid), lambda i: (0, 0)),
        ],
        out_specs=[
            pl.BlockSpec((rb, hid), lambda i: (i, 0)),
            pl.BlockSpec((1, hid), lambda i: (0, 0)),
        ],
        out_shape=[
            jax.ShapeDtypeStruct((n, hid), jnp.float32),
            jax.ShapeDtypeStruct((1, hid), jnp.float32),
        ],
    )


@functools.lru_cache(maxsize=None)
def _make_bn_sum(n, hid, rb):
    def kern(y_ref, st_ref, g_ref, bb_ref, hs_ref):
        i = pl.program_id(0)
        mean = st_ref[0:1] * (1.0 / n)
        var = st_ref[1:2] * (1.0 / n) - mean * mean
        scale = g_ref[...] * lax.rsqrt(var + 1e-5)
        h = jnp.maximum((y_ref[...] - mean) * scale + bb_ref[...], 0.0)

        @pl.when(i == 0)
        def _():
            hs_ref[...] = jnp.zeros_like(hs_ref)

        hs_ref[...] += jnp.sum(h, 0, keepdims=True)

    return pl.pallas_call(
        kern,
        grid=(n // rb,),
        in_specs=[
            pl.BlockSpec((rb, hid), lambda i: (i, 0)),
            pl.BlockSpec((2, hid), lambda i: (0, 0)),
            pl.BlockSpec((1, hid), lambda i: (0, 0)),
            pl.BlockSpec((1, hid), lambda i: (0, 0)),
        ],
        out_specs=pl.BlockSpec((1, hid), lambda i: (0, 0)),
        out_shape=jax.ShapeDtypeStruct((1, hid), jnp.float32),
    )


@functools.lru_cache(maxsize=None)
def _make_colsum(n, d, rb):
    def kern(x_ref, s_ref):
        i = pl.program_id(0)

        @pl.when(i == 0)
        def _():
            s_ref[...] = jnp.zeros_like(s_ref)

        s_ref[...] += jnp.sum(x_ref[...], 0, keepdims=True)

    return pl.pallas_call(
        kern,
        grid=(n // rb,),
        in_specs=[pl.BlockSpec((rb, d), lambda i: (i, 0))],
        out_specs=pl.BlockSpec((1, d), lambda i: (0, 0)),
        out_shape=jax.ShapeDtypeStruct((1, d), jnp.float32),
    )


@functools.lru_cache(maxsize=None)
def _make_final(din, hid, odim, nl):
    def kern(xs_ref, p0_ref, hs_ref, pw_ref, pb_ref, o_ref):
        acc = _dot(xs_ref[...], p0_ref[...])
        for l in range(nl):
            acc = acc + _dot(hs_ref[l:l + 1], pw_ref[l])
        o_ref[...] = acc + jnp.sum(pb_ref[...], 0, keepdims=True)

    return pl.pallas_call(
        kern,
        out_shape=jax.ShapeDtypeStruct((1, odim), jnp.float32),
    )


def kernel(x, edge_index, batch, params):
    n, din = x.shape
    hid = params['convs'][0]['W1'].shape[1]
    odim = params['preds'][0]['W'].shape[1]
    e = edge_index.shape[1]
    nl = len(params['convs'])
    rb = 2000

    nb = -(-e // (_NSUB * _TPB))
    nb += nb & 1
    pad = _NSUB * _TPB * nb - e
    ar = jnp.arange(pad, dtype=jnp.int32)
    src = jnp.concatenate([edge_index[0], n + ar % 96])
    dst = jnp.concatenate([edge_index[1], (ar * 37) % n])
    srcp = src.reshape(_NSUB, nb, _TPB)
    dstp = dst.reshape(_NSUB, nb, _TPB)
    zeros = jnp.zeros((n // _NSUB, 128), jnp.float32)

    hs_list = []
    h = x
    for l in range(nl):
        c = params['convs'][l]
        nch = h.shape[1] // 128
        h3 = h.reshape(n * nch, 128)
        w1r = c['W1'].reshape(nch, 128, hid)
        b1 = c['b1'].reshape(1, hid)
        pooled = _make_segsum(n, nch, nb)(h3, zeros, srcp, dstp)
        m, mst = _make_mm_stats(nch, n, hid, rb)(pooled, h, w1r, b1)
        y, yst = _make_bn_mm(n, hid, rb)(
            m, mst, c['bn1_g'].reshape(1, hid), c['bn1_b'].reshape(1, hid),
            c['W2'], c['b2'].reshape(1, hid))
        if l + 1 < nl:
            h, hsum = _make_bn_relu(n, hid, rb)(
                y, yst, c['bn_g'].reshape(1, hid), c['bn_b'].reshape(1, hid))
        else:
            hsum = _make_bn_sum(n, hid, rb)(
                y, yst, c['bn_g'].reshape(1, hid), c['bn_b'].reshape(1, hid))
        hs_list.append(hsum)

    xsum = _make_colsum(n, din, rb)(x)
    hs = jnp.concatenate(hs_list, 0)
    pw = jnp.stack([params['preds'][l + 1]['W'] for l in range(nl)])
    pb = jnp.stack([params['preds'][l]['b'].reshape(1, odim)
                    for l in range(nl + 1)]).reshape(nl + 1, odim)
    return _make_final(din, hid, odim, nl)(
        xsum, params['preds'][0]['W'], hs, pw, pb)
